# baseline (speedup 1.0000x reference)
import functools

import jax
import jax.numpy as jnp
from jax import lax
from jax.experimental import pallas as pl
from jax.experimental.pallas import tpu as pltpu
from jax.experimental.pallas import tpu_sc as plsc

N_NODES = 10000
N_EDGES = 160000
DIM = 256
HALF = 128
N_GRAPHS = 64
BN_EPS = 1e-5

NC = 2
NS = 16
EDGES_PER_TILE = 10240
CHUNK = 64
NCHUNK = EDGES_PER_TILE // CHUNK
NBUF = 2
NPAIR = NCHUNK // NBUF
ACC_ROWS = 10112
ROWS_PER_TILE = ACC_ROWS // NS
PAD_DST = ACC_ROWS - 1

BLK = 1000
GRID = N_NODES // BLK


def _make_sc_agg():
    mesh = plsc.VectorSubcoreMesh(
        core_axis_name="c", subcore_axis_name="s", num_cores=NC, num_subcores=NS
    )

    @functools.partial(
        pl.kernel,
        out_type=[
            jax.ShapeDtypeStruct((ACC_ROWS, HALF), jnp.float32),
            jax.ShapeDtypeStruct((ACC_ROWS, HALF), jnp.float32),
        ],
        mesh=mesh,
        scratch_types=[
            pltpu.VMEM((EDGES_PER_TILE,), jnp.int32),
            pltpu.VMEM((NCHUNK, CHUNK), jnp.int32),
            pltpu.VMEM((CHUNK, HALF), jnp.float32),
            pltpu.VMEM((CHUNK, HALF), jnp.float32),
            pltpu.VMEM_SHARED((ACC_ROWS, HALF), jnp.float32),
            pltpu.SemaphoreType.DMA((NBUF,)),
            pltpu.SemaphoreType.DMA((NBUF,)),
        ],
    )
    def sc_agg(hlo_hbm, hhi_hbm, src_hbm, dst_hbm, zero_hbm, alo_hbm, ahi_hbm,
               src_v, dst_v, r0, r1, acc_sh, gsem, ssem):
        rows = [r0, r1]
        c = lax.axis_index("c")
        s = lax.axis_index("s")
        pltpu.sync_copy(zero_hbm, acc_sh.at[pl.ds(s * ROWS_PER_TILE, ROWS_PER_TILE)])
        pltpu.sync_copy(src_hbm.at[pl.ds(s * EDGES_PER_TILE, EDGES_PER_TILE)],
                        src_v)
        pltpu.sync_copy(dst_hbm.at[s], dst_v)
        plsc.subcore_barrier()

        def run(h_hbm, out_hbm):
            for k in range(NBUF):
                pltpu.async_copy(
                    h_hbm.at[src_v.at[pl.ds(k * CHUNK, CHUNK)]], rows[k],
                    gsem.at[k])

            @pl.loop(0, NPAIR)
            def _(g):
                base = g * NBUF
                for k in range(NBUF):
                    j = base + k
                    pltpu.make_async_copy(
                        h_hbm.at[src_v.at[pl.ds(0, CHUNK)]], rows[k], gsem.at[k]
                    ).wait()
                    pltpu.async_copy(
                        rows[k], acc_sh.at[dst_v.at[j]], ssem.at[k], add=True)

                    @pl.when(g + 1 < NPAIR)
                    def _():
                        pltpu.make_async_copy(
                            rows[k], acc_sh.at[dst_v.at[0]], ssem.at[k]
                        ).wait()
                        pltpu.async_copy(
                            h_hbm.at[src_v.at[pl.ds((j + NBUF) * CHUNK, CHUNK)]],
                            rows[k], gsem.at[k])

            for k in range(NBUF):
                pltpu.make_async_copy(
                    rows[k], acc_sh.at[dst_v.at[0]], ssem.at[k]
                ).wait()
            plsc.subcore_barrier()
            sl = pl.ds(s * ROWS_PER_TILE, ROWS_PER_TILE)
            pltpu.sync_copy(acc_sh.at[sl], out_hbm.at[sl])

        @pl.when(c == 0)
        def _():
            run(hlo_hbm, alo_hbm)

        @pl.when(c == 1)
        def _():
            run(hhi_hbm, ahi_hbm)

    return sc_agg


_SC_AGG_CACHE = []


def _sc_agg(*args):
    if not _SC_AGG_CACHE:
        _SC_AGG_CACHE.append(_make_sc_agg())
    return _SC_AGG_CACHE[0](*args)


def _mlp_stats_body(hlo, hhi, alo, ahi, w1, b1, w2, b2, m_out, ssum, ssq):
    i = pl.program_id(0)
    h = jnp.concatenate([hlo[...] + alo[...], hhi[...] + ahi[...]], axis=1)
    z = jnp.maximum(
        jnp.dot(h, w1[...], preferred_element_type=jnp.float32) + b1[...], 0.0
    )
    m = jnp.dot(z, w2[...], preferred_element_type=jnp.float32) + b2[...]
    m = jnp.maximum(m, 0.0)
    m_out[...] = m
    cs = jnp.sum(m, axis=0, keepdims=True)
    cq = jnp.sum(m * m, axis=0, keepdims=True)

    @pl.when(i == 0)
    def _():
        ssum[...] = cs
        ssq[...] = cq

    @pl.when(i > 0)
    def _():
        ssum[...] += cs
        ssq[...] += cq


def _mlp_stats(hlo, hhi, alo, ahi, w1, b1, w2, b2):
    half_in = pl.BlockSpec((BLK, HALF), lambda i: (i, 0))
    full_w = pl.BlockSpec((DIM, DIM), lambda i: (0, 0))
    row = pl.BlockSpec((1, DIM), lambda i: (0, 0))
    return pl.pallas_call(
        _mlp_stats_body,
        grid=(GRID,),
        in_specs=[half_in, half_in, half_in, half_in, full_w, row, full_w, row],
        out_specs=[
            pl.BlockSpec((BLK, DIM), lambda i: (i, 0)),
            row,
            row,
        ],
        out_shape=[
            jax.ShapeDtypeStruct((N_NODES, DIM), jnp.float32),
            jax.ShapeDtypeStruct((1, DIM), jnp.float32),
            jax.ShapeDtypeStruct((1, DIM), jnp.float32),
        ],
    )(hlo, hhi, alo, ahi, w1, b1, w2, b2)


def _norm_pool_body(m_ref, ssum, ssq, g_ref, be_ref, batch_ref,
                    hlo_out, hhi_out, pool_out):
    i = pl.program_id(0)
    inv_n = 1.0 / N_NODES
    mean = ssum[...] * inv_n
    var = ssq[...] * inv_n - mean * mean
    scale = g_ref[...] * lax.rsqrt(var + BN_EPS)
    shift = be_ref[...] - mean * scale
    hq = m_ref[...] * scale + shift
    hlo_out[...] = hq[:, :HALF]
    hhi_out[...] = hq[:, HALF:]
    bb = batch_ref[0, 0, :]
    onehot = (bb[None, :] == lax.broadcasted_iota(jnp.int32, (N_GRAPHS, BLK), 0))
    contrib = jnp.dot(onehot.astype(jnp.float32), hq,
                      preferred_element_type=jnp.float32)

    @pl.when(i == 0)
    def _():
        pool_out[...] = contrib

    @pl.when(i > 0)
    def _():
        pool_out[...] += contrib


def _norm_pool(m, ssum, ssq, g, be, batch3d):
    row = pl.BlockSpec((1, DIM), lambda i: (0, 0))
    return pl.pallas_call(
        _norm_pool_body,
        grid=(GRID,),
        in_specs=[
            pl.BlockSpec((BLK, DIM), lambda i: (i, 0)),
            row, row, row, row,
            pl.BlockSpec((1, 1, BLK), lambda i: (i, 0, 0)),
        ],
        out_specs=[
            pl.BlockSpec((BLK, HALF), lambda i: (i, 0)),
            pl.BlockSpec((BLK, HALF), lambda i: (i, 0)),
            pl.BlockSpec((N_GRAPHS, DIM), lambda i: (0, 0)),
        ],
        out_shape=[
            jax.ShapeDtypeStruct((N_NODES, HALF), jnp.float32),
            jax.ShapeDtypeStruct((N_NODES, HALF), jnp.float32),
            jax.ShapeDtypeStruct((N_GRAPHS, DIM), jnp.float32),
        ],
    )(m, ssum, ssq, g, be, batch3d)


def kernel(x, edge_index, batch,
           W1_0, b1_0, W2_0, b2_0, g_0, be_0,
           W1_1, b1_1, W2_1, b2_1, g_1, be_1,
           W1_2, b1_2, W2_2, b2_2, g_2, be_2):
    params = [(W1_0, b1_0, W2_0, b2_0, g_0, be_0),
              (W1_1, b1_1, W2_1, b2_1, g_1, be_1),
              (W1_2, b1_2, W2_2, b2_2, g_2, be_2)]
    n_real = edge_index.shape[1] // NS
    src2 = edge_index[0].reshape(NS, n_real)
    dst2 = edge_index[1].reshape(NS, n_real)
    pad = EDGES_PER_TILE - n_real
    src_r = jnp.concatenate(
        [src2, jnp.zeros((NS, pad), jnp.int32)], axis=1).reshape(-1)
    dst_r = jnp.concatenate(
        [dst2, jnp.full((NS, pad), PAD_DST, jnp.int32)], axis=1
    ).reshape(NS, NCHUNK, CHUNK)
    zeros = jnp.zeros((ROWS_PER_TILE, HALF), jnp.float32)
    batch3d = batch.reshape(GRID, 1, BLK)

    h_lo = x[:, :HALF]
    h_hi = x[:, HALF:]
    halves = []
    pools = []
    for (w1, b1, w2, b2, g, be) in params:
        agg_lo, agg_hi = _sc_agg(h_lo, h_hi, src_r, dst_r, zeros)
        m, ssum, ssq = _mlp_stats(h_lo, h_hi, agg_lo, agg_hi,
                                  w1, b1.reshape(1, DIM), w2, b2.reshape(1, DIM))
        h_lo, h_hi, pool = _norm_pool(m, ssum, ssq, g.reshape(1, DIM),
                                      be.reshape(1, DIM), batch3d)
        halves.extend([h_lo, h_hi])
        pools.append(pool)

    x_nodes = jnp.concatenate(halves, axis=1)
    x_g = jnp.concatenate(pools, axis=1)
    return (x_g, x_nodes)
